# Initial kernel scaffold; baseline (speedup 1.0000x reference)
import jax
import jax.numpy as jnp
from jax.experimental import pallas as pl


def kernel(nodes, edges, globs, adjmat, pre_ln_g, pre_ln_b, feat_W1, feat_b1, feat_W2, feat_b2, attn_W1, attn_b1, attn_W2, attn_b2):
    raise NotImplementedError("write your pallas kernel here")


# speedup vs baseline: 29.7327x; 29.7327x over previous
import math

import jax
import jax.numpy as jnp
from jax.experimental import pallas as pl
from jax.experimental.pallas import tpu as pltpu

B, N = 32, 64
DN, DE, DG = 64, 32, 16
OUT_E = 32
N_HEADS = 4
HEAD_DIM = OUT_E // N_HEADS
HDDN = 64
EDGE_IN = DE + 2 * DN
INV_SQRT_OUT = 1.0 / math.sqrt(OUT_E)


def _edge_block_kernel(nodes_ref, edges_ref, globs_ref, g_ref, beta_ref,
                       fW1_ref, fb1_ref, fW2_ref, fb2_ref,
                       aW1_ref, ab1_ref, aW2r_ref, ab2r_ref,
                       feats_ref, pooled_ref):
    nodes = nodes_ref[0]
    edges = edges_ref[0]
    glb = globs_ref[0]

    send = jnp.broadcast_to(nodes.reshape(N, 1, DN), (N, N, DN)).reshape(N * N, DN)
    recv = jnp.broadcast_to(nodes.reshape(1, N, DN), (N, N, DN)).reshape(N * N, DN)
    e = jnp.concatenate([send, recv, edges], axis=1)

    mu = jnp.mean(e, axis=1, keepdims=True)
    d = e - mu
    var = jnp.mean(d * d, axis=1, keepdims=True)
    eln = d * jax.lax.rsqrt(var + 1e-5) * g_ref[...] + beta_ref[...]

    fW1 = fW1_ref[...]
    aW1 = aW1_ref[...]
    h_f = eln @ fW1[:EDGE_IN] + (glb @ fW1[EDGE_IN:] + fb1_ref[...])
    h_f = h_f * jax.nn.sigmoid(h_f)
    feat_out = h_f @ fW2_ref[...] + fb2_ref[...]
    feats = feat_out + edges

    h_a = eln @ aW1[:EDGE_IN] + (glb @ aW1[EDGE_IN:] + ab1_ref[...])
    h_a = h_a * jax.nn.sigmoid(h_a)
    aw = h_a @ aW2r_ref[...] + ab2r_ref[...]

    aw3 = aw.reshape(N, N, OUT_E)
    m = jnp.max(aw3, axis=0, keepdims=True)
    ex = jnp.exp(aw3 - m)
    w3 = ex / jnp.sum(ex, axis=0, keepdims=True)

    weighted = feats.reshape(N, N, OUT_E) * w3
    pooled = jnp.sum(weighted, axis=0) * INV_SQRT_OUT

    feats_ref[0] = feats
    pooled_ref[0] = pooled


def kernel(nodes, edges, globs, adjmat, pre_ln_g, pre_ln_b,
           feat_W1, feat_b1, feat_W2, feat_b2,
           attn_W1, attn_b1, attn_W2, attn_b2):
    del adjmat
    edges3 = edges.reshape(B, N * N, DE)
    globs3 = globs.reshape(B, 1, DG)
    g2 = pre_ln_g.reshape(1, EDGE_IN)
    beta2 = pre_ln_b.reshape(1, EDGE_IN)
    fb1 = feat_b1.reshape(1, HDDN)
    fb2 = feat_b2.reshape(1, OUT_E)
    ab1 = attn_b1.reshape(1, HDDN)
    aW2r = jnp.repeat(attn_W2, HEAD_DIM, axis=1)
    ab2r = jnp.repeat(attn_b2, HEAD_DIM).reshape(1, OUT_E)

    full = lambda shape: pl.BlockSpec(shape, lambda b: tuple(0 for _ in shape))
    feats, pooled = pl.pallas_call(
        _edge_block_kernel,
        grid=(B,),
        in_specs=[
            pl.BlockSpec((1, N, DN), lambda b: (b, 0, 0)),
            pl.BlockSpec((1, N * N, DE), lambda b: (b, 0, 0)),
            pl.BlockSpec((1, 1, DG), lambda b: (b, 0, 0)),
            full((1, EDGE_IN)),
            full((1, EDGE_IN)),
            full((EDGE_IN + DG, HDDN)),
            full((1, HDDN)),
            full((HDDN, OUT_E)),
            full((1, OUT_E)),
            full((EDGE_IN + DG, HDDN)),
            full((1, HDDN)),
            full((HDDN, OUT_E)),
            full((1, OUT_E)),
        ],
        out_specs=[
            pl.BlockSpec((1, N * N, OUT_E), lambda b: (b, 0, 0)),
            pl.BlockSpec((1, N, OUT_E), lambda b: (b, 0, 0)),
        ],
        out_shape=[
            jax.ShapeDtypeStruct((B, N * N, OUT_E), jnp.float32),
            jax.ShapeDtypeStruct((B, N, OUT_E), jnp.float32),
        ],
        compiler_params=pltpu.CompilerParams(
            dimension_semantics=("parallel",),
        ),
    )(nodes, edges3, globs3, g2, beta2,
      feat_W1, fb1, feat_W2, fb2,
      attn_W1, ab1, aW2r, ab2r)
    return feats.reshape(B * N * N, OUT_E), pooled


# speedup vs baseline: 36.5775x; 1.2302x over previous
import math

import jax
import jax.numpy as jnp
from jax.experimental import pallas as pl
from jax.experimental.pallas import tpu as pltpu

B, N = 32, 64
DN, DE, DG = 64, 32, 16
OUT_E = 32
N_HEADS = 4
HEAD_DIM = OUT_E // N_HEADS
HDDN = 64
EDGE_IN = DE + 2 * DN
H2 = 2 * HDDN
INV_SQRT_OUT = 1.0 / math.sqrt(OUT_E)


def _edge_block_kernel(nodes_ref, edges_ref, globs_ref,
                       Ws_ref, Wr_ref, We_ref, Wc_ref,
                       gw_ref, cvec0_ref, W2B_ref, b2B_ref,
                       feats_ref, pooled_ref):
    nodes = nodes_ref[0]
    edges = edges_ref[0]
    glb = globs_ref[0]
    edges3 = edges.reshape(N, N, DE)

    nsum = jnp.sum(nodes, axis=1)
    nsq = jnp.sum(nodes * nodes, axis=1)
    es = jnp.sum(edges3, axis=2, keepdims=True)
    ess = jnp.sum(edges3 * edges3, axis=2, keepdims=True)
    mu = (nsum.reshape(N, 1, 1) + nsum.reshape(1, N, 1) + es) * (1.0 / EDGE_IN)
    msq = (nsq.reshape(N, 1, 1) + nsq.reshape(1, N, 1) + ess) * (1.0 / EDGE_IN)
    inv = jax.lax.rsqrt(msq - mu * mu + 1e-5)

    S = nodes @ Ws_ref[...]
    R = nodes @ Wr_ref[...]
    Et = (edges @ We_ref[...]).reshape(N, N, H2)
    cvec = (glb @ Wc_ref[...] + cvec0_ref[...]).reshape(1, 1, H2)
    gw = gw_ref[...].reshape(1, 1, H2)

    A = S.reshape(N, 1, H2) + R.reshape(1, N, H2) + Et
    h = (A - mu * gw) * inv + cvec
    h = h * jax.nn.sigmoid(h)

    out = (h.reshape(N * N, H2) @ W2B_ref[...] + b2B_ref[...]).reshape(N, N, 2 * OUT_E)
    feats = out[:, :, :OUT_E] + edges3
    aw = out[:, :, OUT_E:]

    m = jnp.max(aw, axis=0, keepdims=True)
    ex = jnp.exp(aw - m)
    w = ex / jnp.sum(ex, axis=0, keepdims=True)
    pooled = jnp.sum(feats * w, axis=0) * INV_SQRT_OUT

    feats_ref[0] = feats.reshape(N * N, OUT_E)
    pooled_ref[0] = pooled


def kernel(nodes, edges, globs, adjmat, pre_ln_g, pre_ln_b,
           feat_W1, feat_b1, feat_W2, feat_b2,
           attn_W1, attn_b1, attn_W2, attn_b2):
    del adjmat
    edges3 = edges.reshape(B, N * N, DE)
    globs3 = globs.reshape(B, 1, DG)

    W1B = jnp.concatenate([feat_W1, attn_W1], axis=1)
    W1Bg = W1B[:EDGE_IN] * pre_ln_g[:, None]
    Ws = W1Bg[:DN]
    Wr = W1Bg[DN:2 * DN]
    We = W1Bg[2 * DN:]
    Wc = W1B[EDGE_IN:]
    gw = (pre_ln_g @ W1B[:EDGE_IN]).reshape(1, H2)
    b1B = jnp.concatenate([feat_b1, attn_b1])
    cvec0 = (pre_ln_b @ W1B[:EDGE_IN] + b1B).reshape(1, H2)
    aW2r = jnp.repeat(attn_W2, HEAD_DIM, axis=1)
    ab2r = jnp.repeat(attn_b2, HEAD_DIM)
    z = jnp.zeros((HDDN, OUT_E), jnp.float32)
    W2B = jnp.concatenate([
        jnp.concatenate([feat_W2, z], axis=1),
        jnp.concatenate([z, aW2r], axis=1),
    ], axis=0)
    b2B = jnp.concatenate([feat_b2, ab2r]).reshape(1, 2 * OUT_E)

    full = lambda shape: pl.BlockSpec(shape, lambda b: tuple(0 for _ in shape))
    feats, pooled = pl.pallas_call(
        _edge_block_kernel,
        grid=(B,),
        in_specs=[
            pl.BlockSpec((1, N, DN), lambda b: (b, 0, 0)),
            pl.BlockSpec((1, N * N, DE), lambda b: (b, 0, 0)),
            pl.BlockSpec((1, 1, DG), lambda b: (b, 0, 0)),
            full((DN, H2)),
            full((DN, H2)),
            full((DE, H2)),
            full((DG, H2)),
            full((1, H2)),
            full((1, H2)),
            full((H2, 2 * OUT_E)),
            full((1, 2 * OUT_E)),
        ],
        out_specs=[
            pl.BlockSpec((1, N * N, OUT_E), lambda b: (b, 0, 0)),
            pl.BlockSpec((1, N, OUT_E), lambda b: (b, 0, 0)),
        ],
        out_shape=[
            jax.ShapeDtypeStruct((B, N * N, OUT_E), jnp.float32),
            jax.ShapeDtypeStruct((B, N, OUT_E), jnp.float32),
        ],
        compiler_params=pltpu.CompilerParams(
            dimension_semantics=("parallel",),
        ),
    )(nodes, edges3, globs3, Ws, Wr, We, Wc, gw, cvec0, W2B, b2B)
    return feats.reshape(B * N * N, OUT_E), pooled
